# Initial kernel scaffold; baseline (speedup 1.0000x reference)
import jax
import jax.numpy as jnp
from jax.experimental import pallas as pl


def kernel(feat, edge_index_rel0, edge_index_rel1, edge_index_rel2, W0_r0, b0_r0, W1_r0, b1_r0, W0_r1, b0_r1, W1_r1, b1_r1, W0_r2, b0_r2, W1_r2, b1_r2, ln_gamma, ln_beta, att_W, att_b, att_v):
    raise NotImplementedError("write your pallas kernel here")


# speedup vs baseline: 1.4707x; 1.4707x over previous
import functools

import jax
import jax.numpy as jnp
from jax import lax
from jax.experimental import pallas as pl
from jax.experimental.pallas import tpu as pltpu
from jax.experimental.pallas import tpu_sc as plsc

N = 50000
E = 200000
D = 128
DA = 64
R = 3

NC = 2
NS = 16
B = 128
CW = 32
DW = 16
NCHUNK = D // CW
ACC_ROWS = 50048
RPT = ACC_ROWS // NS
EPT = 12544
E_PAD = EPT * NS
STEPS = EPT // B
HSTEPS = STEPS // 2


def _sc_agg(table, gidx_all, dst_all, zrows, ones32):
    mesh = plsc.VectorSubcoreMesh(
        core_axis_name="c", subcore_axis_name="s", num_cores=NC,
        num_subcores=NS)

    @functools.partial(
        pl.kernel,
        out_type=(
            jax.ShapeDtypeStruct((R, 2, NC * ACC_ROWS, CW), jnp.float32),
            jax.ShapeDtypeStruct((R, NC * ACC_ROWS, DW), jnp.float32),
        ),
        mesh=mesh,
        compiler_params=pltpu.CompilerParams(use_tc_tiling_on_sc=False),
        scratch_types=[
            pltpu.VMEM((B,), jnp.int32),
            pltpu.VMEM((B,), jnp.int32),
            pltpu.VMEM((B, CW), jnp.float32),
            pltpu.VMEM((B, CW), jnp.float32),
            pltpu.VMEM((B, CW), jnp.float32),
            pltpu.VMEM_SHARED((ACC_ROWS, CW), jnp.float32),
            pltpu.SemaphoreType.DMA,
        ],
    )
    def agg(table_hbm, gidx_hbm, dst_hbm, z_hbm, ones_hbm,
            sums_out, deg_out,
            gidx_v, dst_v, rows_v, zb, ones_v, acc, sem):
        cid = lax.axis_index("c")
        tid = lax.axis_index("s")
        row_base = tid * RPT

        pltpu.sync_copy(z_hbm, zb)
        pltpu.sync_copy(ones_hbm, ones_v)

        def zero_acc():
            for i in range(RPT // B):
                pltpu.sync_copy(zb, acc.at[pl.ds(row_base + i * B, B)])
            pltpu.sync_copy(zb, acc.at[pl.ds(row_base + RPT - B, B)])

        for p in range(R * 2):
            rel = p // 2
            local = p % 2
            chunk = cid * 2 + local
            gplane = (rel * NCHUNK + chunk) * E_PAD
            ebase = tid * EPT

            zero_acc()
            plsc.subcore_barrier()

            @pl.loop(0, STEPS)
            def step(i):
                off = pl.multiple_of(ebase + i * B, B)
                pltpu.sync_copy(
                    gidx_hbm.at[pl.ds(pl.multiple_of(gplane + off, B), B)],
                    gidx_v)
                pltpu.sync_copy(
                    dst_hbm.at[pl.ds(pl.multiple_of(rel * E_PAD + off, B), B)],
                    dst_v)
                pltpu.async_copy(table_hbm.at[gidx_v], rows_v, sem).wait()
                pltpu.sync_copy(rows_v, acc.at[dst_v], add=True)

            plsc.subcore_barrier()
            pltpu.sync_copy(
                acc.at[pl.ds(row_base, RPT)],
                sums_out.at[rel, local,
                            pl.ds(cid * ACC_ROWS + row_base, RPT)])

        for rel in range(R):
            hbase = rel * E_PAD + cid * (E_PAD // 2) + tid * (EPT // 2)

            zero_acc()
            plsc.subcore_barrier()

            @pl.loop(0, HSTEPS)
            def dstep(i):
                off = pl.multiple_of(hbase + i * B, B)
                pltpu.sync_copy(dst_hbm.at[pl.ds(off, B)], dst_v)
                pltpu.sync_copy(ones_v, acc.at[dst_v], add=True)

            plsc.subcore_barrier()
            pltpu.sync_copy(
                acc.at[pl.ds(row_base, RPT), pl.ds(0, DW)],
                deg_out.at[rel, pl.ds(cid * ACC_ROWS + row_base, RPT)])

    return agg(table, gidx_all, dst_all, zrows, ones32)


BN = 400
GRID = N // BN


def _tc_pass1_body(feat_ref, sums_ref, deg_ref, w0_ref, b0_ref, w1_ref,
                   b1_ref, lng_ref, lnb_ref, attw_ref, attb_ref, attv_ref,
                   h_ref, wpart_ref):
    f = feat_ref[...]
    lng = lng_ref[0]
    lnb = lnb_ref[0]
    attw = attw_ref[...]
    attb = attb_ref[0]
    attv = attv_ref[0]
    parts = []
    for r in range(R):
        dg = deg_ref[r, 0][:, 0:1] + deg_ref[r, 1][:, 0:1]
        inv = 1.0 / jnp.maximum(dg, 1.0)
        w0 = w0_ref[r]
        x = jnp.dot(f, w0, preferred_element_type=jnp.float32)
        for c in range(NCHUNK):
            neigh_c = sums_ref[r, c % 2, c // 2] * inv
            x = x + jnp.dot(neigh_c, w0[c * CW:(c + 1) * CW, :],
                            preferred_element_type=jnp.float32)
        x = jnp.maximum(x + b0_ref[r], 0.0)
        x = jnp.dot(x, w1_ref[r], preferred_element_type=jnp.float32)
        x = jnp.maximum(x + b1_ref[r], 0.0)
        mu = jnp.mean(x, axis=-1, keepdims=True)
        xc = x - mu
        var = jnp.mean(xc * xc, axis=-1, keepdims=True)
        hn = lng * xc / jnp.sqrt(var + 1e-5) + lnb
        h_ref[r] = hn
        t = jnp.tanh(jnp.dot(hn, attw, preferred_element_type=jnp.float32)
                     + attb)
        parts.append(jnp.sum(t * attv, axis=0))

    @pl.when(pl.program_id(0) == 0)
    def _():
        wpart_ref[...] = jnp.zeros((R, DA), jnp.float32)

    wpart_ref[...] += jnp.stack(parts)


def _tc_pass1(feat, sums, deg, w0s, b0s, w1s, b1s, lng, lnb, attw, attb,
              attv):
    return pl.pallas_call(
        _tc_pass1_body,
        grid=(GRID,),
        in_specs=[
            pl.BlockSpec((BN, D), lambda i: (i, 0)),
            pl.BlockSpec((R, 2, NC, BN, CW), lambda i: (0, 0, 0, i, 0)),
            pl.BlockSpec((R, NC, BN, DW), lambda i: (0, 0, i, 0)),
            pl.BlockSpec((R, D, D), lambda i: (0, 0, 0)),
            pl.BlockSpec((R, D), lambda i: (0, 0)),
            pl.BlockSpec((R, D, D), lambda i: (0, 0, 0)),
            pl.BlockSpec((R, D), lambda i: (0, 0)),
            pl.BlockSpec((1, D), lambda i: (0, 0)),
            pl.BlockSpec((1, D), lambda i: (0, 0)),
            pl.BlockSpec((D, DA), lambda i: (0, 0)),
            pl.BlockSpec((1, DA), lambda i: (0, 0)),
            pl.BlockSpec((1, DA), lambda i: (0, 0)),
        ],
        out_specs=[
            pl.BlockSpec((R, BN, D), lambda i: (0, i, 0)),
            pl.BlockSpec((R, DA), lambda i: (0, 0)),
        ],
        out_shape=[
            jax.ShapeDtypeStruct((R, N, D), jnp.float32),
            jax.ShapeDtypeStruct((R, DA), jnp.float32),
        ],
    )(feat, sums, deg, w0s, b0s, w1s, b1s, lng, lnb, attw, attb, attv)


def _tc_pass2_body(beta_ref, h_ref, out_ref):
    out_ref[...] = (beta_ref[0] * h_ref[0] + beta_ref[1] * h_ref[1]
                    + beta_ref[2] * h_ref[2])


def _tc_pass2(beta, h):
    return pl.pallas_call(
        _tc_pass2_body,
        grid=(GRID,),
        in_specs=[
            pl.BlockSpec(memory_space=pltpu.SMEM),
            pl.BlockSpec((R, BN, D), lambda i: (0, i, 0)),
        ],
        out_specs=pl.BlockSpec((BN, D), lambda i: (i, 0)),
        out_shape=jax.ShapeDtypeStruct((N, D), jnp.float32),
    )(beta, h)


def kernel(feat, edge_index_rel0, edge_index_rel1, edge_index_rel2,
           W0_r0, b0_r0, W1_r0, b1_r0,
           W0_r1, b0_r1, W1_r1, b1_r1,
           W0_r2, b0_r2, W1_r2, b1_r2,
           ln_gamma, ln_beta, att_W, att_b, att_v):
    table = jnp.concatenate(
        [feat[:, c * CW:(c + 1) * CW] for c in range(NCHUNK)], axis=0)
    pad = E_PAD - E
    gidxs, dsts = [], []
    for ei in (edge_index_rel0, edge_index_rel1, edge_index_rel2):
        s = jnp.pad(ei[0], (0, pad))
        gidxs.append(jnp.concatenate([s + c * N for c in range(NCHUNK)]))
        dsts.append(jnp.pad(ei[1], (0, pad), constant_values=N))
    gidx_all = jnp.concatenate(gidxs)
    dst_all = jnp.concatenate(dsts)
    zrows = jnp.zeros((B, CW), jnp.float32)
    ones32 = jnp.zeros((B, CW), jnp.float32).at[:, 0].set(1.0)

    sums, deg = _sc_agg(table, gidx_all, dst_all, zrows, ones32)
    sums = sums.reshape(R, 2, NC, ACC_ROWS, CW)
    deg = deg.reshape(R, NC, ACC_ROWS, DW)

    w0s = jnp.stack([W0_r0, W0_r1, W0_r2])
    b0s = jnp.stack([b0_r0, b0_r1, b0_r2])
    w1s = jnp.stack([W1_r0, W1_r1, W1_r2])
    b1s = jnp.stack([b1_r0, b1_r1, b1_r2])

    h, wpart = _tc_pass1(feat, sums, deg, w0s, b0s, w1s, b1s,
                         ln_gamma[None, :], ln_beta[None, :], att_W,
                         att_b[None, :], att_v[None, :])

    w = wpart.sum(axis=1) * (1.0 / N)
    beta = jax.nn.softmax(w)

    return _tc_pass2(beta, h)


# speedup vs baseline: 1.7932x; 1.2193x over previous
import functools

import jax
import jax.numpy as jnp
from jax import lax
from jax.experimental import pallas as pl
from jax.experimental.pallas import tpu as pltpu
from jax.experimental.pallas import tpu_sc as plsc

N = 50000
E = 200000
D = 128
DA = 64
R = 3

NC = 2
NS = 16
B = 128
CW = 32
DW = 16
KG = 4
KD = 5
NCHUNK = D // CW
ACC_ROWS = 50048
RPT = ACC_ROWS // NS
ZROWS = RPT // 2
STEPS_G = 100
EPT_G = STEPS_G * B
E_PAD = EPT_G * NS
HSTEPS = STEPS_G // 2


def _sc_agg(table, gidx_all, dst_all, zrows, ones32):
    mesh = plsc.VectorSubcoreMesh(
        core_axis_name="c", subcore_axis_name="s", num_cores=NC,
        num_subcores=NS)

    @functools.partial(
        pl.kernel,
        out_type=(
            jax.ShapeDtypeStruct((R, 2, NC * ACC_ROWS, CW), jnp.float32),
            jax.ShapeDtypeStruct((R, NC * ACC_ROWS, DW), jnp.float32),
        ),
        mesh=mesh,
        compiler_params=pltpu.CompilerParams(use_tc_tiling_on_sc=False),
        scratch_types=[
            *[pltpu.VMEM((B,), jnp.int32) for _ in range(KG)],
            *[pltpu.VMEM((B,), jnp.int32) for _ in range(KD)],
            pltpu.VMEM((KG, B, CW), jnp.float32),
            pltpu.VMEM((B, CW), jnp.float32),
            pltpu.VMEM_SHARED((ACC_ROWS, CW), jnp.float32),
            pltpu.SemaphoreType.DMA,
            pltpu.SemaphoreType.DMA,
            pltpu.SemaphoreType.DMA,
        ],
    )
    def agg(table_hbm, gidx_hbm, dst_hbm, z_hbm, ones_hbm,
            sums_out, deg_out, *rest):
        gidx_v = list(rest[0:KG])
        dst_v = list(rest[KG:KG + KD])
        bufs, ones_v, acc, isem, gsem, ssem = rest[KG + KD:]
        cid = lax.axis_index("c")
        tid = lax.axis_index("s")
        row_base = tid * RPT

        pltpu.sync_copy(ones_hbm, ones_v)

        def zero_acc():
            d1 = pltpu.async_copy(
                z_hbm, acc.at[pl.ds(row_base, ZROWS)], ssem)
            d2 = pltpu.async_copy(
                z_hbm, acc.at[pl.ds(row_base + ZROWS, ZROWS)], ssem)
            d1.wait()
            d2.wait()

        for p in range(R * 2):
            rel = p // 2
            local = p % 2
            chunk = cid * 2 + local
            gbase0 = ((rel * NCHUNK + chunk) * NS + tid) * EPT_G
            dbase0 = (rel * NS + tid) * EPT_G

            zero_acc()
            plsc.subcore_barrier()

            @pl.loop(0, STEPS_G // KG)
            def step(i):
                base = i * KG
                ids = []
                for j in range(KG):
                    ids.append(pltpu.async_copy(
                        gidx_hbm.at[pl.ds(pl.multiple_of(
                            gbase0 + (base + j) * B, B), B)],
                        gidx_v[j], isem))
                    ids.append(pltpu.async_copy(
                        dst_hbm.at[pl.ds(pl.multiple_of(
                            dbase0 + (base + j) * B, B), B)],
                        dst_v[j], isem))
                for d in ids:
                    d.wait()
                gds = [pltpu.async_copy(
                    table_hbm.at[gidx_v[j]], bufs.at[j], gsem)
                    for j in range(KG)]
                for d in gds:
                    d.wait()
                sds = [pltpu.async_copy(
                    bufs.at[j], acc.at[dst_v[j]], ssem, add=True)
                    for j in range(KG)]
                for d in sds:
                    d.wait()

            plsc.subcore_barrier()
            pltpu.sync_copy(
                acc.at[pl.ds(row_base, RPT)],
                sums_out.at[rel, local,
                            pl.ds(cid * ACC_ROWS + row_base, RPT)])

        for rel in range(R):
            dbase0 = (rel * NS + tid) * EPT_G

            zero_acc()
            plsc.subcore_barrier()

            @pl.loop(0, HSTEPS // KD)
            def dstep(i):
                base = cid * HSTEPS + i * KD
                ids = [pltpu.async_copy(
                    dst_hbm.at[pl.ds(pl.multiple_of(
                        dbase0 + (base + j) * B, B), B)],
                    dst_v[j], isem)
                    for j in range(KD)]
                for d in ids:
                    d.wait()
                sds = [pltpu.async_copy(
                    ones_v, acc.at[dst_v[j]], ssem, add=True)
                    for j in range(KD)]
                for d in sds:
                    d.wait()

            plsc.subcore_barrier()
            pltpu.sync_copy(
                acc.at[pl.ds(row_base, RPT), pl.ds(0, DW)],
                deg_out.at[rel, pl.ds(cid * ACC_ROWS + row_base, RPT)])

    return agg(table, gidx_all, dst_all, zrows, ones32)


BN = 400
GRID = N // BN


def _tc_pass1_body(feat_ref, sums_ref, deg_ref, w0_ref, b0_ref, w1_ref,
                   b1_ref, lng_ref, lnb_ref, attw_ref, attb_ref, attv_ref,
                   h_ref, wpart_ref):
    f = feat_ref[...]
    lng = lng_ref[0]
    lnb = lnb_ref[0]
    attw = attw_ref[...]
    attb = attb_ref[0]
    attv = attv_ref[0]
    parts = []
    for r in range(R):
        dg = deg_ref[r, 0][:, 0:1] + deg_ref[r, 1][:, 0:1]
        inv = 1.0 / jnp.maximum(dg, 1.0)
        w0 = w0_ref[r]
        x = jnp.dot(f, w0, preferred_element_type=jnp.float32)
        for c in range(NCHUNK):
            neigh_c = sums_ref[r, c % 2, c // 2] * inv
            x = x + jnp.dot(neigh_c, w0[c * CW:(c + 1) * CW, :],
                            preferred_element_type=jnp.float32)
        x = jnp.maximum(x + b0_ref[r], 0.0)
        x = jnp.dot(x, w1_ref[r], preferred_element_type=jnp.float32)
        x = jnp.maximum(x + b1_ref[r], 0.0)
        mu = jnp.mean(x, axis=-1, keepdims=True)
        xc = x - mu
        var = jnp.mean(xc * xc, axis=-1, keepdims=True)
        hn = lng * xc / jnp.sqrt(var + 1e-5) + lnb
        h_ref[r] = hn
        t = jnp.tanh(jnp.dot(hn, attw, preferred_element_type=jnp.float32)
                     + attb)
        parts.append(jnp.sum(t * attv, axis=0))

    @pl.when(pl.program_id(0) == 0)
    def _():
        wpart_ref[...] = jnp.zeros((R, DA), jnp.float32)

    wpart_ref[...] += jnp.stack(parts)


def _tc_pass1(feat, sums, deg, w0s, b0s, w1s, b1s, lng, lnb, attw, attb,
              attv):
    return pl.pallas_call(
        _tc_pass1_body,
        grid=(GRID,),
        in_specs=[
            pl.BlockSpec((BN, D), lambda i: (i, 0)),
            pl.BlockSpec((R, 2, NC, BN, CW), lambda i: (0, 0, 0, i, 0)),
            pl.BlockSpec((R, NC, BN, DW), lambda i: (0, 0, i, 0)),
            pl.BlockSpec((R, D, D), lambda i: (0, 0, 0)),
            pl.BlockSpec((R, D), lambda i: (0, 0)),
            pl.BlockSpec((R, D, D), lambda i: (0, 0, 0)),
            pl.BlockSpec((R, D), lambda i: (0, 0)),
            pl.BlockSpec((1, D), lambda i: (0, 0)),
            pl.BlockSpec((1, D), lambda i: (0, 0)),
            pl.BlockSpec((D, DA), lambda i: (0, 0)),
            pl.BlockSpec((1, DA), lambda i: (0, 0)),
            pl.BlockSpec((1, DA), lambda i: (0, 0)),
        ],
        out_specs=[
            pl.BlockSpec((R, BN, D), lambda i: (0, i, 0)),
            pl.BlockSpec((R, DA), lambda i: (0, 0)),
        ],
        out_shape=[
            jax.ShapeDtypeStruct((R, N, D), jnp.float32),
            jax.ShapeDtypeStruct((R, DA), jnp.float32),
        ],
    )(feat, sums, deg, w0s, b0s, w1s, b1s, lng, lnb, attw, attb, attv)


def _tc_pass2_body(beta_ref, h_ref, out_ref):
    out_ref[...] = (beta_ref[0] * h_ref[0] + beta_ref[1] * h_ref[1]
                    + beta_ref[2] * h_ref[2])


def _tc_pass2(beta, h):
    return pl.pallas_call(
        _tc_pass2_body,
        grid=(GRID,),
        in_specs=[
            pl.BlockSpec(memory_space=pltpu.SMEM),
            pl.BlockSpec((R, BN, D), lambda i: (0, i, 0)),
        ],
        out_specs=pl.BlockSpec((BN, D), lambda i: (i, 0)),
        out_shape=jax.ShapeDtypeStruct((N, D), jnp.float32),
    )(beta, h)


def kernel(feat, edge_index_rel0, edge_index_rel1, edge_index_rel2,
           W0_r0, b0_r0, W1_r0, b1_r0,
           W0_r1, b0_r1, W1_r1, b1_r1,
           W0_r2, b0_r2, W1_r2, b1_r2,
           ln_gamma, ln_beta, att_W, att_b, att_v):
    table = jnp.concatenate(
        [feat[:, c * CW:(c + 1) * CW] for c in range(NCHUNK)], axis=0)
    pad = E_PAD - E
    gidxs, dsts = [], []
    for ei in (edge_index_rel0, edge_index_rel1, edge_index_rel2):
        s = jnp.pad(ei[0], (0, pad)).reshape(NS, STEPS_G, B)
        for c in range(NCHUNK):
            gidxs.append(s + c * N)
        dsts.append(jnp.pad(ei[1], (0, pad),
                            constant_values=N).reshape(NS, STEPS_G, B))
    gidx_all = jnp.concatenate(gidxs).reshape(-1)
    dst_all = jnp.concatenate(dsts).reshape(-1)
    zrows = jnp.zeros((ZROWS, CW), jnp.float32)
    ones32 = jnp.zeros((B, CW), jnp.float32).at[:, 0].set(1.0)

    sums, deg = _sc_agg(table, gidx_all, dst_all, zrows, ones32)
    sums = sums.reshape(R, 2, NC, ACC_ROWS, CW)
    deg = deg.reshape(R, NC, ACC_ROWS, DW)

    w0s = jnp.stack([W0_r0, W0_r1, W0_r2])
    b0s = jnp.stack([b0_r0, b0_r1, b0_r2])
    w1s = jnp.stack([W1_r0, W1_r1, W1_r2])
    b1s = jnp.stack([b1_r0, b1_r1, b1_r2])

    h, wpart = _tc_pass1(feat, sums, deg, w0s, b0s, w1s, b1s,
                         ln_gamma[None, :], ln_beta[None, :], att_W,
                         att_b[None, :], att_v[None, :])

    w = wpart.sum(axis=1) * (1.0 / N)
    beta = jax.nn.softmax(w)

    return _tc_pass2(beta, h)
